# baseline (speedup 1.0000x reference)
import functools

import jax
import jax.numpy as jnp
from jax import lax
from jax.experimental import pallas as pl
from jax.experimental.pallas import tpu as pltpu
from jax.experimental.pallas import tpu_sc as plsc

N = 10000
E = 320000
D = 128

NC = 2
NT = 16
EPT = 10240
EP = EPT * NT * NC
NA = N + 8

DCH = 128
DNB = EPT // DCH
ACH = 128
ANB = EPT // ACH

_MESH = plsc.VectorSubcoreMesh(core_axis_name="c", subcore_axis_name="s")


@functools.partial(
    pl.kernel,
    mesh=_MESH,
    out_type=jax.ShapeDtypeStruct((NC, N, D), jnp.float32),
    scratch_types=[
        pltpu.VMEM((DNB, DCH), jnp.int32),
        pltpu.VMEM((DCH, D), jnp.float32),
        pltpu.VMEM_SHARED((NA, D), jnp.float32),
        pltpu.SemaphoreType.DMA,
    ],
)
def _deg_pass(dst2_hbm, ones_hbm, zeros_hbm, degp_hbm, idx_d, ones_v, acc_sh,
              dsem):
    cid = lax.axis_index("c")
    sid = lax.axis_index("s")
    wid = cid * NT + sid

    pltpu.sync_copy(dst2_hbm.at[pl.ds(wid * DNB, DNB)], idx_d)
    pltpu.sync_copy(ones_hbm, ones_v)

    @pl.when(sid < 10)
    def _():
        r = sid * 1000
        pltpu.sync_copy(zeros_hbm.at[pl.ds(r, 1000)], acc_sh.at[pl.ds(r, 1000)])

    plsc.subcore_barrier()

    def body(k, carry):
        for b in range(8):
            pltpu.async_copy(ones_v, acc_sh.at[idx_d.at[k * 8 + b]], dsem,
                             add=True)
        for b in range(8):
            pltpu.make_async_copy(ones_v, acc_sh.at[idx_d.at[0]], dsem).wait()
        return carry

    lax.fori_loop(0, DNB // 8, body, 0)
    plsc.subcore_barrier()

    @pl.when(sid < 10)
    def _():
        r = sid * 1000
        pltpu.sync_copy(acc_sh.at[pl.ds(r, 1000)],
                        degp_hbm.at[cid, pl.ds(r, 1000)])


@functools.partial(
    pl.kernel,
    mesh=_MESH,
    out_type=jax.ShapeDtypeStruct((NC, N, D), jnp.float32),
    scratch_types=[
        pltpu.VMEM((3, ACH), jnp.int32),
        pltpu.VMEM((3, ACH), jnp.int32),
        pltpu.VMEM((3, ACH, D), jnp.float32),
        pltpu.VMEM_SHARED((NA, D), jnp.float32),
        pltpu.SemaphoreType.DMA,
        pltpu.SemaphoreType.DMA,
        pltpu.SemaphoreType.DMA,
        pltpu.SemaphoreType.DMA,
        pltpu.SemaphoreType.DMA,
        pltpu.SemaphoreType.DMA,
    ],
)
def _agg_pass(src1_hbm, dst1_hbm, g_hbm, zeros_hbm, accp_hbm,
              sidx, didx, rows, acc_sh, g0, g1, g2, i0, i1, i2):
    cid = lax.axis_index("c")
    sid = lax.axis_index("s")
    wid = cid * NT + sid
    gsem = (g0, g1, g2)
    isem = (i0, i1, i2)
    ebase = wid * EPT

    @pl.when(sid < 10)
    def _():
        r = sid * 1000
        pltpu.sync_copy(zeros_hbm.at[pl.ds(r, 1000)], acc_sh.at[pl.ds(r, 1000)])

    plsc.subcore_barrier()

    for b in range(2):
        pltpu.sync_copy(src1_hbm.at[pl.ds(ebase + b * ACH, ACH)], sidx.at[b])
        pltpu.sync_copy(dst1_hbm.at[pl.ds(ebase + b * ACH, ACH)], didx.at[b])
        pltpu.async_copy(g_hbm.at[sidx.at[b]], rows.at[b], gsem[b])

    def body(k, carry):
        for b in range(3):
            c = k * 3 + b
            p = (b + 2) % 3
            pltpu.make_async_copy(g_hbm.at[sidx.at[b]], rows.at[b],
                                  gsem[b]).wait()
            pltpu.async_copy(
                src1_hbm.at[pl.ds(ebase + (c + 2) * ACH, ACH)], sidx.at[p],
                isem[p])
            pltpu.async_copy(
                dst1_hbm.at[pl.ds(ebase + (c + 2) * ACH, ACH)], didx.at[p],
                isem[p])
            pltpu.sync_copy(rows.at[b], acc_sh.at[didx.at[b]], add=True)
            pltpu.make_async_copy(
                src1_hbm.at[pl.ds(ebase, ACH)], sidx.at[p], isem[p]).wait()
            pltpu.make_async_copy(
                dst1_hbm.at[pl.ds(ebase, ACH)], didx.at[p], isem[p]).wait()
            pltpu.async_copy(g_hbm.at[sidx.at[p]], rows.at[p], gsem[p])
        return carry

    lax.fori_loop(0, (ANB - 2) // 3, body, 0)

    for c in (ANB - 2, ANB - 1):
        b = c % 3
        pltpu.make_async_copy(g_hbm.at[sidx.at[b]], rows.at[b],
                              gsem[b]).wait()
        pltpu.sync_copy(rows.at[b], acc_sh.at[didx.at[b]], add=True)

    plsc.subcore_barrier()

    @pl.when(sid < 10)
    def _():
        r = sid * 1000
        pltpu.sync_copy(acc_sh.at[pl.ds(r, 1000)],
                        accp_hbm.at[cid, pl.ds(r, 1000)])


_RB = 2000


def _h_body(x_ref, w_ref, h_ref):
    h_ref[...] = jnp.dot(x_ref[...], w_ref[...],
                         preferred_element_type=jnp.float32)


def _mm_body(h_ref, degp_ref, g_ref, dinv_ref):
    deg = degp_ref[0, :, 0:1] + degp_ref[1, :, 0:1] + 1.0
    dinv = lax.rsqrt(deg)
    g_ref[...] = h_ref[...] * dinv
    dinv_ref[...] = dinv


def _fin_body(accp_ref, g_ref, dinv_ref, b_ref, o_ref):
    acc = accp_ref[0] + accp_ref[1] + g_ref[...]
    o_ref[...] = jnp.maximum(acc * dinv_ref[...] + b_ref[...], 0.0)


def kernel(x, edge_index, w1, b1):
    src = edge_index[0]
    dst = edge_index[1]
    pad = EP - E
    src1 = jnp.concatenate([src, jnp.arange(pad, dtype=jnp.int32)])
    dst1 = jnp.concatenate([dst, jnp.full((pad,), N, jnp.int32)])
    dst2 = dst1.reshape(-1, DCH)
    ones = jnp.ones((DCH, D), jnp.float32)
    zeros = jnp.zeros((N, D), jnp.float32)

    h = pl.pallas_call(
        _h_body,
        grid=(N // _RB,),
        in_specs=[
            pl.BlockSpec((_RB, D), lambda i: (i, 0)),
            pl.BlockSpec((D, D), lambda i: (0, 0)),
        ],
        out_specs=pl.BlockSpec((_RB, D), lambda i: (i, 0)),
        out_shape=jax.ShapeDtypeStruct((N, D), jnp.float32),
    )(x, w1)

    degp = _deg_pass(dst2, ones, zeros)

    g, dinv = pl.pallas_call(
        _mm_body,
        grid=(N // _RB,),
        in_specs=[
            pl.BlockSpec((_RB, D), lambda i: (i, 0)),
            pl.BlockSpec((NC, _RB, D), lambda i: (0, i, 0)),
        ],
        out_specs=[
            pl.BlockSpec((_RB, D), lambda i: (i, 0)),
            pl.BlockSpec((_RB, 1), lambda i: (i, 0)),
        ],
        out_shape=[
            jax.ShapeDtypeStruct((N, D), jnp.float32),
            jax.ShapeDtypeStruct((N, 1), jnp.float32),
        ],
    )(h, degp)

    accp = _agg_pass(src1, dst1, g, zeros)

    out = pl.pallas_call(
        _fin_body,
        grid=(N // _RB,),
        in_specs=[
            pl.BlockSpec((NC, _RB, D), lambda i: (0, i, 0)),
            pl.BlockSpec((_RB, D), lambda i: (i, 0)),
            pl.BlockSpec((_RB, 1), lambda i: (i, 0)),
            pl.BlockSpec((1, D), lambda i: (0, 0)),
        ],
        out_specs=pl.BlockSpec((_RB, D), lambda i: (i, 0)),
        out_shape=jax.ShapeDtypeStruct((N, D), jnp.float32),
    )(accp, g, dinv, b1.reshape(1, D))

    return out


# speedup vs baseline: 1.0182x; 1.0182x over previous
import functools

import jax
import jax.numpy as jnp
from jax import lax
from jax.experimental import pallas as pl
from jax.experimental.pallas import tpu as pltpu
from jax.experimental.pallas import tpu_sc as plsc

N = 10000
E = 320000
D = 128

NC = 2
NT = 16
EPT = 10240
EP = EPT * NT * NC
NA = N + 8

DCH = 128
DNB = EPT // DCH
ACH = 128
ANB = EPT // ACH

_MESH = plsc.VectorSubcoreMesh(core_axis_name="c", subcore_axis_name="s")


@functools.partial(
    pl.kernel,
    mesh=_MESH,
    out_type=jax.ShapeDtypeStruct((NC, N, D), jnp.float32),
    scratch_types=[
        pltpu.VMEM((DNB, DCH), jnp.int32),
        pltpu.VMEM((DCH, D), jnp.float32),
        pltpu.VMEM_SHARED((NA, D), jnp.float32),
        pltpu.SemaphoreType.DMA,
    ],
)
def _deg_pass(dst2_hbm, ones_hbm, zeros_hbm, degp_hbm, idx_d, ones_v, acc_sh,
              dsem):
    cid = lax.axis_index("c")
    sid = lax.axis_index("s")
    wid = cid * NT + sid

    pltpu.sync_copy(dst2_hbm.at[pl.ds(wid * DNB, DNB)], idx_d)
    pltpu.sync_copy(ones_hbm, ones_v)

    @pl.when(sid < 10)
    def _():
        r = sid * 1000
        pltpu.sync_copy(zeros_hbm, acc_sh.at[pl.ds(r, 1000)])

    plsc.subcore_barrier()

    def body(k, carry):
        for b in range(8):
            pltpu.async_copy(ones_v, acc_sh.at[idx_d.at[k * 8 + b]], dsem,
                             add=True)
        for b in range(8):
            pltpu.make_async_copy(ones_v, acc_sh.at[idx_d.at[0]], dsem).wait()
        return carry

    lax.fori_loop(0, DNB // 8, body, 0)
    plsc.subcore_barrier()

    @pl.when(sid < 10)
    def _():
        r = sid * 1000
        pltpu.sync_copy(acc_sh.at[pl.ds(r, 1000)],
                        degp_hbm.at[cid, pl.ds(r, 1000)])


@functools.partial(
    pl.kernel,
    mesh=_MESH,
    out_type=jax.ShapeDtypeStruct((NC, N, D), jnp.float32),
    scratch_types=[
        pltpu.VMEM((3, ACH), jnp.int32),
        pltpu.VMEM((3, ACH), jnp.int32),
        pltpu.VMEM((3, ACH, D), jnp.float32),
        pltpu.VMEM_SHARED((NA, D), jnp.float32),
        pltpu.SemaphoreType.DMA,
        pltpu.SemaphoreType.DMA,
        pltpu.SemaphoreType.DMA,
        pltpu.SemaphoreType.DMA,
        pltpu.SemaphoreType.DMA,
        pltpu.SemaphoreType.DMA,
    ],
)
def _agg_pass(src1_hbm, dst1_hbm, g_hbm, zeros_hbm, accp_hbm,
              sidx, didx, rows, acc_sh, g0, g1, g2, i0, i1, i2):
    cid = lax.axis_index("c")
    sid = lax.axis_index("s")
    wid = cid * NT + sid
    gsem = (g0, g1, g2)
    isem = (i0, i1, i2)
    ebase = wid * EPT

    @pl.when(sid < 10)
    def _():
        r = sid * 1000
        pltpu.sync_copy(zeros_hbm, acc_sh.at[pl.ds(r, 1000)])

    plsc.subcore_barrier()

    for b in range(2):
        pltpu.sync_copy(src1_hbm.at[pl.ds(ebase + b * ACH, ACH)], sidx.at[b])
        pltpu.sync_copy(dst1_hbm.at[pl.ds(ebase + b * ACH, ACH)], didx.at[b])
        pltpu.async_copy(g_hbm.at[sidx.at[b]], rows.at[b], gsem[b])

    def body(k, carry):
        for b in range(3):
            c = k * 3 + b
            p = (b + 2) % 3
            pltpu.make_async_copy(g_hbm.at[sidx.at[b]], rows.at[b],
                                  gsem[b]).wait()
            pltpu.async_copy(
                src1_hbm.at[pl.ds(ebase + (c + 2) * ACH, ACH)], sidx.at[p],
                isem[p])
            pltpu.async_copy(
                dst1_hbm.at[pl.ds(ebase + (c + 2) * ACH, ACH)], didx.at[p],
                isem[p])
            pltpu.sync_copy(rows.at[b], acc_sh.at[didx.at[b]], add=True)
            pltpu.make_async_copy(
                src1_hbm.at[pl.ds(ebase, ACH)], sidx.at[p], isem[p]).wait()
            pltpu.make_async_copy(
                dst1_hbm.at[pl.ds(ebase, ACH)], didx.at[p], isem[p]).wait()
            pltpu.async_copy(g_hbm.at[sidx.at[p]], rows.at[p], gsem[p])
        return carry

    lax.fori_loop(0, (ANB - 2) // 3, body, 0)

    for c in (ANB - 2, ANB - 1):
        b = c % 3
        pltpu.make_async_copy(g_hbm.at[sidx.at[b]], rows.at[b],
                              gsem[b]).wait()
        pltpu.sync_copy(rows.at[b], acc_sh.at[didx.at[b]], add=True)

    plsc.subcore_barrier()

    @pl.when(sid < 10)
    def _():
        r = sid * 1000
        pltpu.sync_copy(acc_sh.at[pl.ds(r, 1000)],
                        accp_hbm.at[cid, pl.ds(r, 1000)])


_RB = 2000


def _mm_body(x_ref, w_ref, degp_ref, g_ref, dinv_ref):
    h = jnp.dot(x_ref[...], w_ref[...], preferred_element_type=jnp.float32)
    deg = degp_ref[0, :, 0:1] + degp_ref[1, :, 0:1] + 1.0
    dinv = lax.rsqrt(deg)
    g_ref[...] = h * dinv
    dinv_ref[...] = dinv


def _fin_body(accp_ref, g_ref, dinv_ref, b_ref, o_ref):
    acc = accp_ref[0] + accp_ref[1] + g_ref[...]
    o_ref[...] = jnp.maximum(acc * dinv_ref[...] + b_ref[...], 0.0)


def kernel(x, edge_index, w1, b1):
    src = edge_index[0]
    dst = edge_index[1]
    pad = EP - E
    src1 = jnp.concatenate([src, jnp.arange(pad, dtype=jnp.int32)])
    dst1 = jnp.concatenate([dst, jnp.full((pad,), N, jnp.int32)])
    dst2 = dst1.reshape(-1, DCH)
    ones = jnp.ones((DCH, D), jnp.float32)
    zeros = jnp.zeros((1000, D), jnp.float32)

    degp = _deg_pass(dst2, ones, zeros)

    g, dinv = pl.pallas_call(
        _mm_body,
        grid=(N // _RB,),
        in_specs=[
            pl.BlockSpec((_RB, D), lambda i: (i, 0)),
            pl.BlockSpec((D, D), lambda i: (0, 0)),
            pl.BlockSpec((NC, _RB, D), lambda i: (0, i, 0)),
        ],
        out_specs=[
            pl.BlockSpec((_RB, D), lambda i: (i, 0)),
            pl.BlockSpec((_RB, 1), lambda i: (i, 0)),
        ],
        out_shape=[
            jax.ShapeDtypeStruct((N, D), jnp.float32),
            jax.ShapeDtypeStruct((N, 1), jnp.float32),
        ],
    )(x, w1, degp)

    accp = _agg_pass(src1, dst1, g, zeros)

    out = pl.pallas_call(
        _fin_body,
        grid=(N // _RB,),
        in_specs=[
            pl.BlockSpec((NC, _RB, D), lambda i: (0, i, 0)),
            pl.BlockSpec((_RB, D), lambda i: (i, 0)),
            pl.BlockSpec((_RB, 1), lambda i: (i, 0)),
            pl.BlockSpec((1, D), lambda i: (0, 0)),
        ],
        out_specs=pl.BlockSpec((_RB, D), lambda i: (i, 0)),
        out_shape=jax.ShapeDtypeStruct((N, D), jnp.float32),
    )(accp, g, dinv, b1.reshape(1, D))

    return out
